# Initial kernel scaffold; baseline (speedup 1.0000x reference)
import jax
import jax.numpy as jnp
from jax.experimental import pallas as pl


def kernel(x, edge_index, W, b):
    raise NotImplementedError("write your pallas kernel here")


# speedup vs baseline: 21.6754x; 21.6754x over previous
import functools

import jax
import jax.numpy as jnp
from jax import lax
from jax.experimental import pallas as pl
from jax.experimental.pallas import tpu as pltpu
from jax.experimental.pallas import tpu_sc as plsc

_NC = 2
_NS = 16
_NW = _NC * _NS
_CH = 128


def _sc_degree(col, n_nodes):
    e = col.shape[0]
    epw = e // _NW
    nfull = epw // _CH
    rem = epw - nfull * _CH
    rpt = 8 * (n_nodes // (8 * _NS))
    tail = n_nodes - rpt * _NS
    tbase = rpt * _NS

    mesh = plsc.VectorSubcoreMesh(core_axis_name="c", subcore_axis_name="s", num_cores=_NC, num_subcores=_NS)

    @functools.partial(
        pl.kernel,
        out_type=jax.ShapeDtypeStruct((_NC, n_nodes, 16), jnp.float32),
        mesh=mesh,
        scratch_types=[
            pltpu.VMEM_SHARED((n_nodes, 16), jnp.float32),
            pltpu.VMEM((_CH,), jnp.int32),
            pltpu.VMEM((rem,), jnp.int32),
            pltpu.VMEM((_CH, 16), jnp.float32),
            pltpu.VMEM((rem, 16), jnp.float32),
            pltpu.VMEM((rpt, 16), jnp.float32),
            pltpu.VMEM((tail, 16), jnp.float32),
        ],
    )
    def k(col_hbm, hist_hbm, acc, idx_v, idx_r, ones_v, ones_r, buf_v, tail_v):
        c = lax.axis_index("c")
        s = lax.axis_index("s")
        wid = s * _NC + c

        def fill(ref, nrows, val):
            def body(i, _):
                ref[i] = jnp.full((16,), val, jnp.float32)
                return 0

            lax.fori_loop(0, nrows, body, 0)

        fill(ones_v, _CH, 1.0)
        fill(ones_r, rem, 1.0)
        fill(buf_v, rpt, 0.0)
        pltpu.sync_copy(buf_v, acc.at[pl.ds(s * rpt, rpt)])

        @pl.when(s == _NS - 1)
        def _():
            pltpu.sync_copy(buf_v.at[pl.ds(0, tail)], acc.at[pl.ds(tbase, tail)])

        plsc.subcore_barrier()

        base = wid * epw

        def chunk(ch, _):
            pltpu.sync_copy(col_hbm.at[pl.ds(base + ch * _CH, _CH)], idx_v)
            pltpu.sync_copy(ones_v, acc.at[idx_v], add=True)
            return 0

        lax.fori_loop(0, nfull, chunk, 0)
        pltpu.sync_copy(col_hbm.at[pl.ds(base + nfull * _CH, rem)], idx_r)
        pltpu.sync_copy(ones_r, acc.at[idx_r], add=True)
        plsc.subcore_barrier()

        pltpu.sync_copy(acc.at[pl.ds(s * rpt, rpt)], buf_v)
        pltpu.sync_copy(buf_v, hist_hbm.at[c, pl.ds(s * rpt, rpt)])

        @pl.when(s == _NS - 1)
        def _():
            pltpu.sync_copy(acc.at[pl.ds(tbase, tail)], tail_v)
            pltpu.sync_copy(tail_v, hist_hbm.at[c, pl.ds(tbase, tail)])

    return k(col)


def _sc_scatter(g, row, col, n_nodes, d):
    e = row.shape[0]
    epw = e // _NW
    nfull = epw // _CH
    rem = epw - nfull * _CH
    rpt = 8 * (n_nodes // (8 * _NS))
    tail = n_nodes - rpt * _NS
    tbase = rpt * _NS
    wb = 104
    nwb = rpt // wb

    mesh = plsc.VectorSubcoreMesh(core_axis_name="c", subcore_axis_name="s", num_cores=_NC, num_subcores=_NS)

    @functools.partial(
        pl.kernel,
        out_type=jax.ShapeDtypeStruct((_NC, n_nodes, d), jnp.float32),
        mesh=mesh,
        scratch_types=[
            pltpu.VMEM_SHARED((n_nodes, d), jnp.float32),
            pltpu.VMEM((_CH,), jnp.int32),
            pltpu.VMEM((_CH,), jnp.int32),
            pltpu.VMEM((rem,), jnp.int32),
            pltpu.VMEM((rem,), jnp.int32),
            pltpu.VMEM((_CH, d), jnp.float32),
            pltpu.VMEM((rem, d), jnp.float32),
            pltpu.VMEM((wb, d), jnp.float32),
            pltpu.VMEM((tail, d), jnp.float32),
        ],
    )
    def k(g_hbm, row_hbm, col_hbm, out_hbm, acc, ri_v, ci_v, ri_r, ci_r,
          rows_v, rows_r, buf_v, tail_v):
        c = lax.axis_index("c")
        s = lax.axis_index("s")
        wid = s * _NC + c

        def zrow(i, _):
            for j in range(d // 16):
                buf_v[i, pl.ds(j * 16, 16)] = jnp.zeros((16,), jnp.float32)
            return 0

        lax.fori_loop(0, wb, zrow, 0)
        for j in range(nwb):
            pltpu.sync_copy(buf_v, acc.at[pl.ds(s * rpt + j * wb, wb)])

        @pl.when(s == _NS - 1)
        def _():
            pltpu.sync_copy(buf_v.at[pl.ds(0, tail)], acc.at[pl.ds(tbase, tail)])

        plsc.subcore_barrier()

        base = wid * epw

        def chunk(ch, _):
            eb = base + ch * _CH
            pltpu.sync_copy(row_hbm.at[pl.ds(eb, _CH)], ri_v)
            pltpu.sync_copy(col_hbm.at[pl.ds(eb, _CH)], ci_v)
            pltpu.sync_copy(g_hbm.at[ri_v], rows_v)
            pltpu.sync_copy(rows_v, acc.at[ci_v], add=True)
            return 0

        lax.fori_loop(0, nfull, chunk, 0)
        eb = base + nfull * _CH
        pltpu.sync_copy(row_hbm.at[pl.ds(eb, rem)], ri_r)
        pltpu.sync_copy(col_hbm.at[pl.ds(eb, rem)], ci_r)
        pltpu.sync_copy(g_hbm.at[ri_r], rows_r)
        pltpu.sync_copy(rows_r, acc.at[ci_r], add=True)
        plsc.subcore_barrier()

        for j in range(nwb):
            pltpu.sync_copy(acc.at[pl.ds(s * rpt + j * wb, wb)], buf_v)
            pltpu.sync_copy(buf_v, out_hbm.at[c, pl.ds(s * rpt + j * wb, wb)])

        @pl.when(s == _NS - 1)
        def _():
            pltpu.sync_copy(acc.at[pl.ds(tbase, tail)], tail_v)
            pltpu.sync_copy(tail_v, out_hbm.at[c, pl.ds(tbase, tail)])

    return k(g, row, col)


def _tc_transform(x, w, hist):
    n, d_in = x.shape
    d_out = w.shape[1]
    blk = 1000

    def body(x_ref, w_ref, h_ref, g_ref):
        deg = (h_ref[0] + h_ref[1])[:, 0:1] + 1.0
        dinv = lax.rsqrt(deg)
        h = jnp.dot(x_ref[...], w_ref[...], preferred_element_type=jnp.float32)
        g_ref[...] = h * dinv

    return pl.pallas_call(
        body,
        grid=(n // blk,),
        in_specs=[
            pl.BlockSpec((blk, d_in), lambda i: (i, 0)),
            pl.BlockSpec((d_in, d_out), lambda i: (0, 0)),
            pl.BlockSpec((2, blk, 16), lambda i: (0, i, 0)),
        ],
        out_specs=pl.BlockSpec((blk, d_out), lambda i: (i, 0)),
        out_shape=jax.ShapeDtypeStruct((n, d_out), jnp.float32),
    )(x, w, hist)


def _tc_finish(accp, g, hist, b):
    n, d = g.shape
    blk = 1000
    b2 = b.reshape(1, d)

    def body(a_ref, g_ref, h_ref, b_ref, o_ref):
        deg = (h_ref[0] + h_ref[1])[:, 0:1] + 1.0
        dinv = lax.rsqrt(deg)
        s = a_ref[0] + a_ref[1] + g_ref[...]
        o_ref[...] = s * dinv + b_ref[...]

    return pl.pallas_call(
        body,
        grid=(n // blk,),
        in_specs=[
            pl.BlockSpec((2, blk, d), lambda i: (0, i, 0)),
            pl.BlockSpec((blk, d), lambda i: (i, 0)),
            pl.BlockSpec((2, blk, 16), lambda i: (0, i, 0)),
            pl.BlockSpec((1, d), lambda i: (0, 0)),
        ],
        out_specs=pl.BlockSpec((blk, d), lambda i: (i, 0)),
        out_shape=jax.ShapeDtypeStruct((n, d), jnp.float32),
    )(accp, g, hist, b2)


def kernel(x, edge_index, W, b):
    n = x.shape[0]
    d = W.shape[1]
    row = edge_index[0]
    col = edge_index[1]
    hist = _sc_degree(col, n)
    g = _tc_transform(x, W, hist)
    accp = _sc_scatter(g, row, col, n, d)
    return _tc_finish(accp, g, hist, b)
